# baseline (speedup 1.0000x reference)
import jax
import jax.numpy as jnp
from jax import lax
from jax.experimental import pallas as pl
from jax.experimental.pallas import tpu as pltpu
from jax.experimental.pallas import tpu_sc as plsc

_N = 10000
_D = 128
_H = _D // 2
_E = 320000
_NP = 10240
_NC = 2
_NS = 16
_KC = 128
_CPT = 160
_CH = _NS * _CPT
_EP = _CH * _KC
_NB = 4
_LA = 2
_RPT = _NP // _NS


def _make_sc_agg(with_deg: bool):
    out_type = [jax.ShapeDtypeStruct((_NC, _NP, _H), jnp.float32)]
    scratch = [
        pltpu.VMEM((_CPT, _KC), jnp.int32),
        pltpu.VMEM((_CPT, _KC), jnp.int32),
        [pltpu.VMEM((_KC, _H), jnp.float32)] * _NB,
        [pltpu.SemaphoreType.DMA] * _NB,
        [pltpu.SemaphoreType.DMA] * _NB,
        pltpu.SemaphoreType.DMA,
        pltpu.VMEM_SHARED((_NP, _H), jnp.float32),
    ]
    if with_deg:
        out_type.append(jax.ShapeDtypeStruct((_NC, _NP), jnp.float32))
        scratch.append(pltpu.VMEM_SHARED((_NP,), jnp.float32))
        scratch.append(pltpu.VMEM((_KC,), jnp.float32))
        scratch.append(pltpu.VMEM((_RPT,), jnp.float32))
    mesh = plsc.VectorSubcoreMesh(core_axis_name="c", subcore_axis_name="s")

    def body(x0_hbm, x1_hbm, src_hbm, dst_hbm, out_hbm, *rest):
        if with_deg:
            (deg_hbm, srcl, dstl, ring, gsems, ssems, isem, agg_sh,
             deg_sh, onesv, zcol) = rest
        else:
            srcl, dstl, ring, gsems, ssems, isem, agg_sh = rest
        cid = lax.axis_index("c")
        sid = lax.axis_index("s")
        zero16 = jnp.zeros((16,), jnp.float32)
        ones16 = jnp.ones((16,), jnp.float32)

        crow = sid * _CPT
        pltpu.async_copy(src_hbm.at[pl.ds(crow, _CPT), :], srcl, isem)
        pltpu.async_copy(dst_hbm.at[pl.ds(crow, _CPT), :], dstl, isem)

        def zrow(i, carry):
            ring[0][i // (_H // 16), pl.ds((i % (_H // 16)) * 16, 16)] = zero16
            return carry
        lax.fori_loop(0, _KC * (_H // 16), zrow, 0)
        base = sid * _RPT
        for r in range(_RPT // _KC):
            pltpu.sync_copy(ring[0], agg_sh.at[pl.ds(base + r * _KC, _KC)])
        if with_deg:
            def zdeg(i, carry):
                zcol[pl.ds(i * 16, 16)] = zero16
                return carry
            lax.fori_loop(0, _RPT // 16, zdeg, 0)

            def fones(i, carry):
                onesv[pl.ds(i * 16, 16)] = ones16
                return carry
            lax.fori_loop(0, _KC // 16, fones, 0)
            pltpu.sync_copy(zcol, deg_sh.at[pl.ds(base, _RPT)])
        pltpu.make_async_copy(src_hbm.at[pl.ds(crow, _CPT), :], srcl,
                              isem).wait()
        pltpu.make_async_copy(dst_hbm.at[pl.ds(crow, _CPT), :], dstl,
                              isem).wait()
        plsc.subcore_barrier()

        def run(xtab, do_deg):
            for b in range(_LA):
                pltpu.async_copy(xtab.at[srcl.at[b]], ring[b], gsems[b])

            def step(i, carry):
                for b in range(_NB):
                    cc = i * _NB + b
                    pltpu.make_async_copy(xtab.at[srcl.at[0]], ring[b],
                                          gsems[b]).wait()
                    pltpu.async_copy(ring[b], agg_sh.at[dstl.at[cc]],
                                     ssems[b], add=True)
                    if do_deg:
                        pltpu.sync_copy(onesv, deg_sh.at[dstl.at[cc]],
                                        add=True)
                    nc = cc + _LA
                    nb = (b + _LA) % _NB

                    @pl.when(cc >= _NB - _LA)
                    def _():
                        pltpu.make_async_copy(
                            ring[nb], agg_sh.at[dstl.at[0]],
                            ssems[nb]).wait()

                    @pl.when(nc < _CPT)
                    def _():
                        pltpu.async_copy(xtab.at[srcl.at[nc]], ring[nb],
                                         gsems[nb])
                return carry
            lax.fori_loop(0, _CPT // _NB, step, 0)
            for k in range(_LA):
                b = (_CPT - _LA + k) % _NB
                pltpu.make_async_copy(ring[b], agg_sh.at[dstl.at[0]],
                                      ssems[b]).wait()

        pl.when(cid == 0)(lambda: run(x0_hbm, with_deg))
        pl.when(cid == 1)(lambda: run(x1_hbm, False))

        plsc.subcore_barrier()
        pltpu.sync_copy(agg_sh.at[pl.ds(base, _RPT)],
                        out_hbm.at[cid, pl.ds(base, _RPT), :])
        if with_deg:
            pltpu.sync_copy(deg_sh.at[pl.ds(base, _RPT)],
                            deg_hbm.at[cid, pl.ds(base, _RPT)])

    return pl.kernel(body, out_type=out_type, mesh=mesh,
                     scratch_types=scratch,
                     compiler_params=pltpu.CompilerParams(
                         use_tc_tiling_on_sc=False))


_SC_AGG_DEG = _make_sc_agg(True)
_SC_AGG = _make_sc_agg(False)

_BN = 1000


def _make_dense(relu: bool, split_out: bool):
    def body(p0, p1, dt, x0r, x1r, wlr, wrr, br, *outs):
        deg = jnp.sum(dt[...], axis=1, keepdims=True)
        inv = 1.0 / jnp.maximum(deg, 1.0)
        agg = jnp.concatenate([p0[0, :, :], p1[0, :, :]], axis=1) * inv
        xr = jnp.concatenate([x0r[...], x1r[...]], axis=1)
        h = (jnp.dot(agg, wlr[...], preferred_element_type=jnp.float32,
                     precision=lax.Precision.HIGHEST)
             + jnp.dot(xr, wrr[...], preferred_element_type=jnp.float32,
                       precision=lax.Precision.HIGHEST)
             + br[...])
        if relu:
            h = jnp.maximum(h, 0.0)
        h = h + xr
        if split_out:
            outs[0][...] = h[:, :_H]
            outs[1][...] = h[:, _H:]
        else:
            outs[0][...] = h

    if split_out:
        out_shape = [jax.ShapeDtypeStruct((_N, _H), jnp.float32)] * 2
        out_specs = [pl.BlockSpec((_BN, _H), lambda i: (i, 0))] * 2
    else:
        out_shape = jax.ShapeDtypeStruct((_N, _D), jnp.float32)
        out_specs = pl.BlockSpec((_BN, _D), lambda i: (i, 0))
    return pl.pallas_call(
        body,
        grid=(_N // _BN,),
        in_specs=[
            pl.BlockSpec((1, _BN, _H), lambda i: (0, i, 0)),
            pl.BlockSpec((1, _BN, _H), lambda i: (1, i, 0)),
            pl.BlockSpec((_BN, _NC), lambda i: (i, 0)),
            pl.BlockSpec((_BN, _H), lambda i: (i, 0)),
            pl.BlockSpec((_BN, _H), lambda i: (i, 0)),
            pl.BlockSpec((_D, _D), lambda i: (0, 0)),
            pl.BlockSpec((_D, _D), lambda i: (0, 0)),
            pl.BlockSpec((1, _D), lambda i: (0, 0)),
        ],
        out_specs=out_specs,
        out_shape=out_shape,
    )


_DENSE_RELU = _make_dense(True, True)
_DENSE_LAST = _make_dense(False, False)


def kernel(x, edge_index, W_l0, W_r0, b0, W_l1, W_r1, b1, W_l2, W_r2, b2):
    npad = _EP - _E
    src2d = jnp.concatenate(
        [edge_index[0], jnp.zeros((npad,), jnp.int32)]).reshape(_CH, _KC)
    dst2d = jnp.concatenate(
        [edge_index[1], jnp.full((npad,), _N, jnp.int32)]).reshape(_CH, _KC)
    x0 = x[:, :_H]
    x1 = x[:, _H:]
    agg, degp = _SC_AGG_DEG(x0, x1, src2d, dst2d)
    degp_t = degp.T
    h0, h1 = _DENSE_RELU(agg, agg, degp_t, x0, x1, W_l0, W_r0,
                         b0.reshape(1, _D))
    agg, = _SC_AGG(h0, h1, src2d, dst2d)
    h0, h1 = _DENSE_RELU(agg, agg, degp_t, h0, h1, W_l1, W_r1,
                         b1.reshape(1, _D))
    agg, = _SC_AGG(h0, h1, src2d, dst2d)
    h = _DENSE_LAST(agg, agg, degp_t, h0, h1, W_l2, W_r2,
                    b2.reshape(1, _D))
    return h


# speedup vs baseline: 1.5974x; 1.5974x over previous
import jax
import jax.numpy as jnp
from jax import lax
from jax.experimental import pallas as pl
from jax.experimental.pallas import tpu as pltpu
from jax.experimental.pallas import tpu_sc as plsc

_N = 10000
_D = 128
_Q = _D // 4
_E = 320000
_NP = 10240
_NC = 2
_NS = 16
_KC = 128
_CPT = 160
_CH = _NS * _CPT
_EP = _CH * _KC
_NB = 4
_LA = 2
_RPT = _NP // _NS


def _make_sc_agg(with_deg: bool):
    out_type = [jax.ShapeDtypeStruct((4, _NP, _Q), jnp.float32)]
    scratch = [
        pltpu.VMEM((_CPT, _KC), jnp.int32),
        pltpu.VMEM((_CPT, _KC), jnp.int32),
        [pltpu.VMEM((_KC, _Q), jnp.float32)] * _NB,
        [pltpu.SemaphoreType.DMA] * _NB,
        [pltpu.SemaphoreType.DMA] * _NB,
        pltpu.SemaphoreType.DMA,
        pltpu.VMEM_SHARED((_NP, _Q), jnp.float32),
        pltpu.VMEM_SHARED((_NP, _Q), jnp.float32),
    ]
    if with_deg:
        out_type.append(jax.ShapeDtypeStruct((_NC, _NP), jnp.float32))
        scratch.append(pltpu.VMEM_SHARED((_NP,), jnp.float32))
        scratch.append(pltpu.VMEM((_KC,), jnp.float32))
        scratch.append(pltpu.VMEM((_RPT,), jnp.float32))
    mesh = plsc.VectorSubcoreMesh(core_axis_name="c", subcore_axis_name="s")

    def body(xq0, xq1, xq2, xq3, src_hbm, dst_hbm, out_hbm, *rest):
        if with_deg:
            (deg_hbm, srcl, dstl, ring, gsems, ssems, isem, tab_sh, agg_sh,
             deg_sh, onesv, zcol) = rest
        else:
            srcl, dstl, ring, gsems, ssems, isem, tab_sh, agg_sh = rest
        cid = lax.axis_index("c")
        sid = lax.axis_index("s")
        zero16 = jnp.zeros((16,), jnp.float32)
        ones16 = jnp.ones((16,), jnp.float32)
        base = sid * _RPT

        crow = sid * _CPT
        pltpu.async_copy(src_hbm.at[pl.ds(crow, _CPT), :], srcl, isem)
        pltpu.async_copy(dst_hbm.at[pl.ds(crow, _CPT), :], dstl, isem)

        if with_deg:
            def zdeg(i, carry):
                zcol[pl.ds(i * 16, 16)] = zero16
                return carry
            lax.fori_loop(0, _RPT // 16, zdeg, 0)

            def fones(i, carry):
                onesv[pl.ds(i * 16, 16)] = ones16
                return carry
            lax.fori_loop(0, _KC // 16, fones, 0)
            pltpu.sync_copy(zcol, deg_sh.at[pl.ds(base, _RPT)])
        pltpu.make_async_copy(src_hbm.at[pl.ds(crow, _CPT), :], srcl,
                              isem).wait()
        pltpu.make_async_copy(dst_hbm.at[pl.ds(crow, _CPT), :], dstl,
                              isem).wait()

        def stage(xq):
            @pl.when(sid < _NS - 1)
            def _():
                pltpu.sync_copy(xq.at[pl.ds(sid * _RPT, _RPT), :],
                                tab_sh.at[pl.ds(sid * _RPT, _RPT)])

            @pl.when(sid == _NS - 1)
            def _():
                pltpu.sync_copy(
                    xq.at[pl.ds((_NS - 1) * _RPT, _N - (_NS - 1) * _RPT), :],
                    tab_sh.at[pl.ds((_NS - 1) * _RPT,
                                    _N - (_NS - 1) * _RPT)])

        def run(do_deg):
            for b in range(_LA):
                pltpu.async_copy(tab_sh.at[srcl.at[b]], ring[b], gsems[b])

            def step(i, carry):
                for b in range(_NB):
                    cc = i * _NB + b
                    pltpu.make_async_copy(tab_sh.at[srcl.at[0]], ring[b],
                                          gsems[b]).wait()
                    pltpu.async_copy(ring[b], agg_sh.at[dstl.at[cc]],
                                     ssems[b], add=True)
                    if do_deg:
                        pltpu.sync_copy(onesv, deg_sh.at[dstl.at[cc]],
                                        add=True)
                    nc = cc + _LA
                    nb = (b + _LA) % _NB

                    @pl.when(cc >= _NB - _LA)
                    def _():
                        pltpu.make_async_copy(
                            ring[nb], agg_sh.at[dstl.at[0]],
                            ssems[nb]).wait()

                    @pl.when(nc < _CPT)
                    def _():
                        pltpu.async_copy(tab_sh.at[srcl.at[nc]], ring[nb],
                                         gsems[nb])
                return carry
            lax.fori_loop(0, _CPT // _NB, step, 0)
            for k in range(_LA):
                b = (_CPT - _LA + k) % _NB
                pltpu.make_async_copy(ring[b], agg_sh.at[dstl.at[0]],
                                      ssems[b]).wait()

        for p in range(2):
            if p == 0:
                pl.when(cid == 0)(lambda: stage(xq0))
                pl.when(cid == 1)(lambda: stage(xq1))
            else:
                pl.when(cid == 0)(lambda: stage(xq2))
                pl.when(cid == 1)(lambda: stage(xq3))
            def zrow(i, carry):
                ring[0][i // (_Q // 16),
                        pl.ds((i % (_Q // 16)) * 16, 16)] = zero16
                return carry
            lax.fori_loop(0, _KC * (_Q // 16), zrow, 0)
            for r in range(_RPT // _KC):
                pltpu.sync_copy(ring[0],
                                agg_sh.at[pl.ds(base + r * _KC, _KC)])
            plsc.subcore_barrier()
            run(with_deg and p == 0)
            plsc.subcore_barrier()
            pltpu.sync_copy(agg_sh.at[pl.ds(base, _RPT)],
                            out_hbm.at[2 * p + cid, pl.ds(base, _RPT), :])
            if with_deg and p == 0:
                pltpu.sync_copy(deg_sh.at[pl.ds(base, _RPT)],
                                deg_hbm.at[cid, pl.ds(base, _RPT)])

    return pl.kernel(body, out_type=out_type, mesh=mesh,
                     scratch_types=scratch,
                     compiler_params=pltpu.CompilerParams(
                         use_tc_tiling_on_sc=False))


_SC_AGG_DEG = _make_sc_agg(True)
_SC_AGG = _make_sc_agg(False)

_BN = 1000


def _make_dense(relu: bool, split_out: bool):
    def body(p0, p1, p2, p3, dt, x0r, x1r, x2r, x3r, wlr, wrr, br, *outs):
        deg = 0.5 * jnp.sum(dt[...], axis=1, keepdims=True)
        inv = 1.0 / jnp.maximum(deg, 1.0)
        agg = jnp.concatenate(
            [p0[0, :, :], p1[0, :, :], p2[0, :, :], p3[0, :, :]],
            axis=1) * inv
        xr = jnp.concatenate(
            [x0r[...], x1r[...], x2r[...], x3r[...]], axis=1)
        h = (jnp.dot(agg, wlr[...], preferred_element_type=jnp.float32,
                     precision=lax.Precision.HIGHEST)
             + jnp.dot(xr, wrr[...], preferred_element_type=jnp.float32,
                       precision=lax.Precision.HIGHEST)
             + br[...])
        if relu:
            h = jnp.maximum(h, 0.0)
        h = h + xr
        if split_out:
            for q in range(4):
                outs[q][...] = h[:, q * _Q:(q + 1) * _Q]
        else:
            outs[0][...] = h

    if split_out:
        out_shape = [jax.ShapeDtypeStruct((_N, _Q), jnp.float32)] * 4
        out_specs = [pl.BlockSpec((_BN, _Q), lambda i: (i, 0))] * 4
    else:
        out_shape = jax.ShapeDtypeStruct((_N, _D), jnp.float32)
        out_specs = pl.BlockSpec((_BN, _D), lambda i: (i, 0))
    qspec = [
        pl.BlockSpec((1, _BN, _Q), lambda i: (0, i, 0)),
        pl.BlockSpec((1, _BN, _Q), lambda i: (1, i, 0)),
        pl.BlockSpec((1, _BN, _Q), lambda i: (2, i, 0)),
        pl.BlockSpec((1, _BN, _Q), lambda i: (3, i, 0)),
    ]
    return pl.pallas_call(
        body,
        grid=(_N // _BN,),
        in_specs=qspec + [
            pl.BlockSpec((_BN, _NC), lambda i: (i, 0)),
            pl.BlockSpec((_BN, _Q), lambda i: (i, 0)),
            pl.BlockSpec((_BN, _Q), lambda i: (i, 0)),
            pl.BlockSpec((_BN, _Q), lambda i: (i, 0)),
            pl.BlockSpec((_BN, _Q), lambda i: (i, 0)),
            pl.BlockSpec((_D, _D), lambda i: (0, 0)),
            pl.BlockSpec((_D, _D), lambda i: (0, 0)),
            pl.BlockSpec((1, _D), lambda i: (0, 0)),
        ],
        out_specs=out_specs,
        out_shape=out_shape,
    )


_DENSE_RELU = _make_dense(True, True)
_DENSE_LAST = _make_dense(False, False)


def kernel(x, edge_index, W_l0, W_r0, b0, W_l1, W_r1, b1, W_l2, W_r2, b2):
    npad = _EP - _E
    src2d = jnp.concatenate(
        [edge_index[0], jnp.zeros((npad,), jnp.int32)]).reshape(_CH, _KC)
    dst2d = jnp.concatenate(
        [edge_index[1], jnp.full((npad,), _N, jnp.int32)]).reshape(_CH, _KC)
    xq = [x[:, q * _Q:(q + 1) * _Q] for q in range(4)]
    agg, degp = _SC_AGG_DEG(*xq, src2d, dst2d)
    degp_t = degp.T
    h = _DENSE_RELU(agg, agg, agg, agg, degp_t, *xq, W_l0, W_r0,
                    b0.reshape(1, _D))
    agg, = _SC_AGG(*h, src2d, dst2d)
    h = _DENSE_RELU(agg, agg, agg, agg, degp_t, *h, W_l1, W_r1,
                    b1.reshape(1, _D))
    agg, = _SC_AGG(*h, src2d, dst2d)
    h = _DENSE_LAST(agg, agg, agg, agg, degp_t, *h, W_l2, W_r2,
                    b2.reshape(1, _D))
    return h


# speedup vs baseline: 1.6104x; 1.0082x over previous
import jax
import jax.numpy as jnp
from jax import lax
from jax.experimental import pallas as pl
from jax.experimental.pallas import tpu as pltpu
from jax.experimental.pallas import tpu_sc as plsc

_N = 10000
_D = 128
_Q = _D // 4
_E = 320000
_NP = 10240
_NC = 2
_NS = 16
_KC = 128
_CPT = 160
_CH = _NS * _CPT
_EP = _CH * _KC
_NB = 8
_LA = 4
_RPT = _NP // _NS


def _make_sc_agg(with_deg: bool):
    out_type = [jax.ShapeDtypeStruct((4, _NP, _Q), jnp.float32)]
    scratch = [
        pltpu.VMEM((_CPT, _KC), jnp.int32),
        pltpu.VMEM((_CPT, _KC), jnp.int32),
        [pltpu.VMEM((_KC, _Q), jnp.float32)] * _NB,
        [pltpu.SemaphoreType.DMA] * _NB,
        [pltpu.SemaphoreType.DMA] * _NB,
        pltpu.SemaphoreType.DMA,
        pltpu.VMEM_SHARED((_NP, _Q), jnp.float32),
        pltpu.VMEM_SHARED((_NP, _Q), jnp.float32),
    ]
    if with_deg:
        out_type.append(jax.ShapeDtypeStruct((_NC, _NP), jnp.float32))
        scratch.append(pltpu.VMEM_SHARED((_NP,), jnp.float32))
        scratch.append(pltpu.VMEM((_KC,), jnp.float32))
        scratch.append(pltpu.VMEM((_RPT,), jnp.float32))
    mesh = plsc.VectorSubcoreMesh(core_axis_name="c", subcore_axis_name="s")

    def body(xq0, xq1, xq2, xq3, src_hbm, dst_hbm, out_hbm, *rest):
        if with_deg:
            (deg_hbm, srcl, dstl, ring, gsems, ssems, isem, tab_sh, agg_sh,
             deg_sh, onesv, zcol) = rest
        else:
            srcl, dstl, ring, gsems, ssems, isem, tab_sh, agg_sh = rest
        cid = lax.axis_index("c")
        sid = lax.axis_index("s")
        zero16 = jnp.zeros((16,), jnp.float32)
        ones16 = jnp.ones((16,), jnp.float32)
        base = sid * _RPT

        crow = sid * _CPT
        pltpu.async_copy(src_hbm.at[pl.ds(crow, _CPT), :], srcl, isem)
        pltpu.async_copy(dst_hbm.at[pl.ds(crow, _CPT), :], dstl, isem)

        if with_deg:
            def zdeg(i, carry):
                zcol[pl.ds(i * 16, 16)] = zero16
                return carry
            lax.fori_loop(0, _RPT // 16, zdeg, 0)

            def fones(i, carry):
                onesv[pl.ds(i * 16, 16)] = ones16
                return carry
            lax.fori_loop(0, _KC // 16, fones, 0)
            pltpu.sync_copy(zcol, deg_sh.at[pl.ds(base, _RPT)])
        pltpu.make_async_copy(src_hbm.at[pl.ds(crow, _CPT), :], srcl,
                              isem).wait()
        pltpu.make_async_copy(dst_hbm.at[pl.ds(crow, _CPT), :], dstl,
                              isem).wait()

        def stage(xq):
            @pl.when(sid < _NS - 1)
            def _():
                pltpu.sync_copy(xq.at[pl.ds(sid * _RPT, _RPT), :],
                                tab_sh.at[pl.ds(sid * _RPT, _RPT)])

            @pl.when(sid == _NS - 1)
            def _():
                pltpu.sync_copy(
                    xq.at[pl.ds((_NS - 1) * _RPT, _N - (_NS - 1) * _RPT), :],
                    tab_sh.at[pl.ds((_NS - 1) * _RPT,
                                    _N - (_NS - 1) * _RPT)])

        def run(do_deg):
            for b in range(_LA):
                pltpu.async_copy(tab_sh.at[srcl.at[b]], ring[b], gsems[b])

            def step(i, carry):
                for b in range(_NB):
                    cc = i * _NB + b
                    pltpu.make_async_copy(tab_sh.at[srcl.at[0]], ring[b],
                                          gsems[b]).wait()
                    pltpu.async_copy(ring[b], agg_sh.at[dstl.at[cc]],
                                     ssems[b], add=True)
                    if do_deg:
                        pltpu.sync_copy(onesv, deg_sh.at[dstl.at[cc]],
                                        add=True)
                    nc = cc + _LA
                    nb = (b + _LA) % _NB

                    @pl.when(cc >= _NB - _LA)
                    def _():
                        pltpu.make_async_copy(
                            ring[nb], agg_sh.at[dstl.at[0]],
                            ssems[nb]).wait()

                    @pl.when(nc < _CPT)
                    def _():
                        pltpu.async_copy(tab_sh.at[srcl.at[nc]], ring[nb],
                                         gsems[nb])
                return carry
            lax.fori_loop(0, _CPT // _NB, step, 0)
            for k in range(_LA):
                b = (_CPT - _LA + k) % _NB
                pltpu.make_async_copy(ring[b], agg_sh.at[dstl.at[0]],
                                      ssems[b]).wait()

        for p in range(2):
            if p == 0:
                pl.when(cid == 0)(lambda: stage(xq0))
                pl.when(cid == 1)(lambda: stage(xq1))
            else:
                pl.when(cid == 0)(lambda: stage(xq2))
                pl.when(cid == 1)(lambda: stage(xq3))
            def zrow(i, carry):
                ring[0][i // (_Q // 16),
                        pl.ds((i % (_Q // 16)) * 16, 16)] = zero16
                return carry
            lax.fori_loop(0, _KC * (_Q // 16), zrow, 0)
            for r in range(_RPT // _KC):
                pltpu.sync_copy(ring[0],
                                agg_sh.at[pl.ds(base + r * _KC, _KC)])
            plsc.subcore_barrier()
            run(with_deg and p == 0)
            plsc.subcore_barrier()
            pltpu.sync_copy(agg_sh.at[pl.ds(base, _RPT)],
                            out_hbm.at[2 * p + cid, pl.ds(base, _RPT), :])
            if with_deg and p == 0:
                pltpu.sync_copy(deg_sh.at[pl.ds(base, _RPT)],
                                deg_hbm.at[cid, pl.ds(base, _RPT)])

    return pl.kernel(body, out_type=out_type, mesh=mesh,
                     scratch_types=scratch,
                     compiler_params=pltpu.CompilerParams(
                         use_tc_tiling_on_sc=False))


_SC_AGG_DEG = _make_sc_agg(True)
_SC_AGG = _make_sc_agg(False)

_BN = 1000


def _make_dense(relu: bool, split_out: bool):
    def body(p0, p1, p2, p3, dt, x0r, x1r, x2r, x3r, wlr, wrr, br, *outs):
        deg = 0.5 * jnp.sum(dt[...], axis=1, keepdims=True)
        inv = 1.0 / jnp.maximum(deg, 1.0)
        agg = jnp.concatenate(
            [p0[0, :, :], p1[0, :, :], p2[0, :, :], p3[0, :, :]],
            axis=1) * inv
        xr = jnp.concatenate(
            [x0r[...], x1r[...], x2r[...], x3r[...]], axis=1)
        h = (jnp.dot(agg, wlr[...], preferred_element_type=jnp.float32,
                     precision=lax.Precision.HIGHEST)
             + jnp.dot(xr, wrr[...], preferred_element_type=jnp.float32,
                       precision=lax.Precision.HIGHEST)
             + br[...])
        if relu:
            h = jnp.maximum(h, 0.0)
        h = h + xr
        if split_out:
            for q in range(4):
                outs[q][...] = h[:, q * _Q:(q + 1) * _Q]
        else:
            outs[0][...] = h

    if split_out:
        out_shape = [jax.ShapeDtypeStruct((_N, _Q), jnp.float32)] * 4
        out_specs = [pl.BlockSpec((_BN, _Q), lambda i: (i, 0))] * 4
    else:
        out_shape = jax.ShapeDtypeStruct((_N, _D), jnp.float32)
        out_specs = pl.BlockSpec((_BN, _D), lambda i: (i, 0))
    qspec = [
        pl.BlockSpec((1, _BN, _Q), lambda i: (0, i, 0)),
        pl.BlockSpec((1, _BN, _Q), lambda i: (1, i, 0)),
        pl.BlockSpec((1, _BN, _Q), lambda i: (2, i, 0)),
        pl.BlockSpec((1, _BN, _Q), lambda i: (3, i, 0)),
    ]
    return pl.pallas_call(
        body,
        grid=(_N // _BN,),
        in_specs=qspec + [
            pl.BlockSpec((_BN, _NC), lambda i: (i, 0)),
            pl.BlockSpec((_BN, _Q), lambda i: (i, 0)),
            pl.BlockSpec((_BN, _Q), lambda i: (i, 0)),
            pl.BlockSpec((_BN, _Q), lambda i: (i, 0)),
            pl.BlockSpec((_BN, _Q), lambda i: (i, 0)),
            pl.BlockSpec((_D, _D), lambda i: (0, 0)),
            pl.BlockSpec((_D, _D), lambda i: (0, 0)),
            pl.BlockSpec((1, _D), lambda i: (0, 0)),
        ],
        out_specs=out_specs,
        out_shape=out_shape,
    )


_DENSE_RELU = _make_dense(True, True)
_DENSE_LAST = _make_dense(False, False)


def kernel(x, edge_index, W_l0, W_r0, b0, W_l1, W_r1, b1, W_l2, W_r2, b2):
    npad = _EP - _E
    src2d = jnp.concatenate(
        [edge_index[0], jnp.zeros((npad,), jnp.int32)]).reshape(_CH, _KC)
    dst2d = jnp.concatenate(
        [edge_index[1], jnp.full((npad,), _N, jnp.int32)]).reshape(_CH, _KC)
    xq = [x[:, q * _Q:(q + 1) * _Q] for q in range(4)]
    agg, degp = _SC_AGG_DEG(*xq, src2d, dst2d)
    degp_t = degp.T
    h = _DENSE_RELU(agg, agg, agg, agg, degp_t, *xq, W_l0, W_r0,
                    b0.reshape(1, _D))
    agg, = _SC_AGG(*h, src2d, dst2d)
    h = _DENSE_RELU(agg, agg, agg, agg, degp_t, *h, W_l1, W_r1,
                    b1.reshape(1, _D))
    agg, = _SC_AGG(*h, src2d, dst2d)
    h = _DENSE_LAST(agg, agg, agg, agg, degp_t, *h, W_l2, W_r2,
                    b2.reshape(1, _D))
    return h


# speedup vs baseline: 1.7108x; 1.0623x over previous
import jax
import jax.numpy as jnp
from jax import lax
from jax.experimental import pallas as pl
from jax.experimental.pallas import tpu as pltpu
from jax.experimental.pallas import tpu_sc as plsc

_N = 10000
_D = 128
_H = _D // 2
_E = 320000
_NA = 10016
_NC = 2
_NS = 16
_KC = 128
_CPT = 160
_CH = _NS * _CPT
_EP = _CH * _KC
_NB = 4
_LA = 2
_RF = 632
_RL = _N - 15 * _RF


def _make_sc_agg(with_deg: bool):
    out_type = [jax.ShapeDtypeStruct((_NC, _N, _H), jnp.float32)]
    scratch = [
        pltpu.VMEM((_CPT // 4, _KC), jnp.int32),
        pltpu.VMEM((_CPT // 4, _KC), jnp.int32),
        [pltpu.VMEM((_KC, _H), jnp.float32)] * _NB,
        [pltpu.SemaphoreType.DMA] * _NB,
        [pltpu.SemaphoreType.DMA] * _NB,
        pltpu.SemaphoreType.DMA,
        pltpu.VMEM_SHARED((_N, _H), jnp.float32),
        pltpu.VMEM_SHARED((_NA, _H), jnp.float32),
    ]
    if with_deg:
        out_type.append(jax.ShapeDtypeStruct((_NC, _N), jnp.float32))
        scratch.append(pltpu.VMEM_SHARED((_NA,), jnp.float32))
        scratch.append(pltpu.VMEM((_KC,), jnp.float32))
        scratch.append(pltpu.VMEM((_RF + 8,), jnp.float32))
    mesh = plsc.VectorSubcoreMesh(core_axis_name="c", subcore_axis_name="s")

    def body(x0_hbm, x1_hbm, src_hbm, dst_hbm, out_hbm, *rest):
        if with_deg:
            (deg_hbm, srcl, dstl, ring, gsems, ssems, isem, tab_sh, agg_sh,
             deg_sh, onesv, zcol) = rest
        else:
            srcl, dstl, ring, gsems, ssems, isem, tab_sh, agg_sh = rest
        cid = lax.axis_index("c")
        sid = lax.axis_index("s")
        zero16 = jnp.zeros((16,), jnp.float32)
        ones16 = jnp.ones((16,), jnp.float32)
        base = sid * _RF

        crow = sid * _CPT
        def stage(xq):
            @pl.when(sid < _NS - 1)
            def _():
                pltpu.sync_copy(xq.at[pl.ds(base, _RF), :],
                                tab_sh.at[pl.ds(base, _RF)])

            @pl.when(sid == _NS - 1)
            def _():
                pltpu.sync_copy(xq.at[pl.ds(15 * _RF, _RL), :],
                                tab_sh.at[pl.ds(15 * _RF, _RL)])

        pl.when(cid == 0)(lambda: stage(x0_hbm))
        pl.when(cid == 1)(lambda: stage(x1_hbm))

        def zrow(i, carry):
            ring[0][i // (_H // 16), pl.ds((i % (_H // 16)) * 16, 16)] = zero16
            return carry
        lax.fori_loop(0, _KC * (_H // 16), zrow, 0)
        for r in range(4):
            pltpu.sync_copy(ring[0], agg_sh.at[pl.ds(base + r * _KC, _KC)])

        @pl.when(sid < _NS - 1)
        def _():
            pltpu.sync_copy(ring[0].at[pl.ds(0, _RF - 4 * _KC)],
                            agg_sh.at[pl.ds(base + 4 * _KC, _RF - 4 * _KC)])

        @pl.when(sid == _NS - 1)
        def _():
            pltpu.sync_copy(
                ring[0].at[pl.ds(0, _RL + 16 - 4 * _KC)],
                agg_sh.at[pl.ds(base + 4 * _KC, _RL + 16 - 4 * _KC)])

        if with_deg:
            def zdeg(i, carry):
                zcol[pl.ds(i * 16, 16)] = zero16
                return carry
            lax.fori_loop(0, (_RF + 8) // 16, zdeg, 0)

            def fones(i, carry):
                onesv[pl.ds(i * 16, 16)] = ones16
                return carry
            lax.fori_loop(0, _KC // 16, fones, 0)

            @pl.when(sid < _NS - 1)
            def _():
                pltpu.sync_copy(zcol.at[pl.ds(0, _RF)], deg_sh.at[pl.ds(base, _RF)])

            @pl.when(sid == _NS - 1)
            def _():
                pltpu.sync_copy(zcol.at[pl.ds(0, _RL + 16)],
                                deg_sh.at[pl.ds(15 * _RF, _RL + 16)])

        plsc.subcore_barrier()

        _QC = _CPT // 4
        for q in range(4):
            pltpu.sync_copy(src_hbm.at[pl.ds(crow + q * _QC, _QC), :], srcl)
            pltpu.sync_copy(dst_hbm.at[pl.ds(crow + q * _QC, _QC), :], dstl)
            for b in range(_LA):
                pltpu.async_copy(tab_sh.at[srcl.at[b]], ring[b], gsems[b])

            def step(i, carry):
                for b in range(_NB):
                    cc = i * _NB + b
                    pltpu.make_async_copy(tab_sh.at[srcl.at[0]], ring[b],
                                          gsems[b]).wait()
                    pltpu.async_copy(ring[b], agg_sh.at[dstl.at[cc]],
                                     ssems[b], add=True)
                    if with_deg:
                        pltpu.sync_copy(onesv, deg_sh.at[dstl.at[cc]],
                                        add=True)
                    nc = cc + _LA
                    nb = (b + _LA) % _NB

                    @pl.when(cc >= _NB - _LA)
                    def _():
                        pltpu.make_async_copy(ring[nb],
                                              agg_sh.at[dstl.at[0]],
                                              ssems[nb]).wait()

                    @pl.when(nc < _QC)
                    def _():
                        pltpu.async_copy(tab_sh.at[srcl.at[nc]], ring[nb],
                                         gsems[nb])
                return carry
            lax.fori_loop(0, _QC // _NB, step, 0)
            for k in range(_LA):
                b = (_QC - _LA + k) % _NB
                pltpu.make_async_copy(ring[b], agg_sh.at[dstl.at[0]],
                                      ssems[b]).wait()

        plsc.subcore_barrier()

        @pl.when(sid < _NS - 1)
        def _():
            pltpu.sync_copy(agg_sh.at[pl.ds(base, _RF)],
                            out_hbm.at[cid, pl.ds(base, _RF), :])

        @pl.when(sid == _NS - 1)
        def _():
            pltpu.sync_copy(agg_sh.at[pl.ds(15 * _RF, _RL)],
                            out_hbm.at[cid, pl.ds(15 * _RF, _RL), :])

        if with_deg:
            @pl.when(sid < _NS - 1)
            def _():
                pltpu.sync_copy(deg_sh.at[pl.ds(base, _RF)],
                                deg_hbm.at[cid, pl.ds(base, _RF)])

            @pl.when(sid == _NS - 1)
            def _():
                pltpu.sync_copy(deg_sh.at[pl.ds(15 * _RF, _RL)],
                                deg_hbm.at[cid, pl.ds(15 * _RF, _RL)])

    return pl.kernel(body, out_type=out_type, mesh=mesh,
                     scratch_types=scratch,
                     compiler_params=pltpu.CompilerParams(
                         use_tc_tiling_on_sc=False))


_SC_AGG_DEG = _make_sc_agg(True)
_SC_AGG = _make_sc_agg(False)

_BN = 1000


def _make_dense(relu: bool, split_out: bool):
    def body(p0, p1, dt, x0r, x1r, wlr, wrr, br, *outs):
        deg = 0.5 * jnp.sum(dt[...], axis=1, keepdims=True)
        inv = 1.0 / jnp.maximum(deg, 1.0)
        agg = jnp.concatenate([p0[0, :, :], p1[0, :, :]], axis=1) * inv
        xr = jnp.concatenate([x0r[...], x1r[...]], axis=1)
        h = (jnp.dot(agg, wlr[...], preferred_element_type=jnp.float32,
                     precision=lax.Precision.HIGHEST)
             + jnp.dot(xr, wrr[...], preferred_element_type=jnp.float32,
                       precision=lax.Precision.HIGHEST)
             + br[...])
        if relu:
            h = jnp.maximum(h, 0.0)
        h = h + xr
        if split_out:
            outs[0][...] = h[:, :_H]
            outs[1][...] = h[:, _H:]
        else:
            outs[0][...] = h

    if split_out:
        out_shape = [jax.ShapeDtypeStruct((_N, _H), jnp.float32)] * 2
        out_specs = [pl.BlockSpec((_BN, _H), lambda i: (i, 0))] * 2
    else:
        out_shape = jax.ShapeDtypeStruct((_N, _D), jnp.float32)
        out_specs = pl.BlockSpec((_BN, _D), lambda i: (i, 0))
    return pl.pallas_call(
        body,
        grid=(_N // _BN,),
        in_specs=[
            pl.BlockSpec((1, _BN, _H), lambda i: (0, i, 0)),
            pl.BlockSpec((1, _BN, _H), lambda i: (1, i, 0)),
            pl.BlockSpec((_BN, _NC), lambda i: (i, 0)),
            pl.BlockSpec((_BN, _H), lambda i: (i, 0)),
            pl.BlockSpec((_BN, _H), lambda i: (i, 0)),
            pl.BlockSpec((_D, _D), lambda i: (0, 0)),
            pl.BlockSpec((_D, _D), lambda i: (0, 0)),
            pl.BlockSpec((1, _D), lambda i: (0, 0)),
        ],
        out_specs=out_specs,
        out_shape=out_shape,
    )


_DENSE_RELU = _make_dense(True, True)
_DENSE_LAST = _make_dense(False, False)


def kernel(x, edge_index, W_l0, W_r0, b0, W_l1, W_r1, b1, W_l2, W_r2, b2):
    npad = _EP - _E
    src2d = jnp.concatenate(
        [edge_index[0], jnp.zeros((npad,), jnp.int32)]).reshape(_CH, _KC)
    dst2d = jnp.concatenate(
        [edge_index[1], jnp.full((npad,), _N, jnp.int32)]).reshape(_CH, _KC)
    x0 = x[:, :_H]
    x1 = x[:, _H:]
    agg, degp = _SC_AGG_DEG(x0, x1, src2d, dst2d)
    degp_t = degp.T
    h0, h1 = _DENSE_RELU(agg, agg, degp_t, x0, x1, W_l0, W_r0,
                         b0.reshape(1, _D))
    agg, = _SC_AGG(h0, h1, src2d, dst2d)
    h0, h1 = _DENSE_RELU(agg, agg, degp_t, h0, h1, W_l1, W_r1,
                         b1.reshape(1, _D))
    agg, = _SC_AGG(h0, h1, src2d, dst2d)
    h = _DENSE_LAST(agg, agg, degp_t, h0, h1, W_l2, W_r2,
                    b2.reshape(1, _D))
    return h


# speedup vs baseline: 1.8552x; 1.0845x over previous
import jax
import jax.numpy as jnp
from jax import lax
from jax.experimental import pallas as pl
from jax.experimental.pallas import tpu as pltpu
from jax.experimental.pallas import tpu_sc as plsc

_N = 10000
_D = 128
_H = _D // 2
_E = 320000
_NA = 10016
_NC = 2
_NS = 16
_KC = 128
_CPT = 160
_CH = _NS * _CPT
_EP = _CH * _KC
_NB = 4
_LA = 2
_RF = 632
_RL = _N - 15 * _RF


def _make_sc_agg(with_deg: bool):
    out_type = [jax.ShapeDtypeStruct((_NC, _N, _H), jnp.float32)]
    scratch = [
        pltpu.VMEM((_CPT // 4, _KC), jnp.int32),
        pltpu.VMEM((_CPT // 4, _KC), jnp.int32),
        [pltpu.VMEM((_KC, _H), jnp.float32)] * _NB,
        [pltpu.SemaphoreType.DMA] * _NB,
        [pltpu.SemaphoreType.DMA] * _NB,
        pltpu.SemaphoreType.DMA,
        pltpu.VMEM_SHARED((_N, _H), jnp.float32),
        pltpu.VMEM_SHARED((_NA, _H), jnp.float32),
    ]
    if with_deg:
        out_type.append(jax.ShapeDtypeStruct((_NC, _N), jnp.float32))
        scratch.append(pltpu.VMEM_SHARED((_NA,), jnp.float32))
        scratch.append(pltpu.VMEM((_KC,), jnp.float32))
        scratch.append(pltpu.SemaphoreType.DMA)
        scratch.append(pltpu.VMEM((_RF + 8,), jnp.float32))
    mesh = plsc.VectorSubcoreMesh(core_axis_name="c", subcore_axis_name="s")

    def body(x0_hbm, x1_hbm, src_hbm, dst_hbm, out_hbm, *rest):
        if with_deg:
            (deg_hbm, srcl, dstl, ring, gsems, ssems, isem, tab_sh, agg_sh,
             deg_sh, onesv, dsem, zcol) = rest
        else:
            srcl, dstl, ring, gsems, ssems, isem, tab_sh, agg_sh = rest
        cid = lax.axis_index("c")
        sid = lax.axis_index("s")
        zero16 = jnp.zeros((16,), jnp.float32)
        ones16 = jnp.ones((16,), jnp.float32)
        base = sid * _RF

        crow = sid * _CPT
        def stage(xq):
            @pl.when(sid < _NS - 1)
            def _():
                pltpu.sync_copy(xq.at[pl.ds(base, _RF), :],
                                tab_sh.at[pl.ds(base, _RF)])

            @pl.when(sid == _NS - 1)
            def _():
                pltpu.sync_copy(xq.at[pl.ds(15 * _RF, _RL), :],
                                tab_sh.at[pl.ds(15 * _RF, _RL)])

        pl.when(cid == 0)(lambda: stage(x0_hbm))
        pl.when(cid == 1)(lambda: stage(x1_hbm))

        def zrow(i, carry):
            ring[0][i // (_H // 16), pl.ds((i % (_H // 16)) * 16, 16)] = zero16
            return carry
        lax.fori_loop(0, _KC * (_H // 16), zrow, 0)
        for r in range(4):
            pltpu.sync_copy(ring[0], agg_sh.at[pl.ds(base + r * _KC, _KC)])

        @pl.when(sid < _NS - 1)
        def _():
            pltpu.sync_copy(ring[0].at[pl.ds(0, _RF - 4 * _KC)],
                            agg_sh.at[pl.ds(base + 4 * _KC, _RF - 4 * _KC)])

        @pl.when(sid == _NS - 1)
        def _():
            pltpu.sync_copy(
                ring[0].at[pl.ds(0, _RL + 16 - 4 * _KC)],
                agg_sh.at[pl.ds(base + 4 * _KC, _RL + 16 - 4 * _KC)])

        if with_deg:
            def zdeg(i, carry):
                zcol[pl.ds(i * 16, 16)] = zero16
                return carry
            lax.fori_loop(0, (_RF + 8) // 16, zdeg, 0)

            def fones(i, carry):
                onesv[pl.ds(i * 16, 16)] = ones16
                return carry
            lax.fori_loop(0, _KC // 16, fones, 0)

            @pl.when(sid < _NS - 1)
            def _():
                pltpu.sync_copy(zcol.at[pl.ds(0, _RF)], deg_sh.at[pl.ds(base, _RF)])

            @pl.when(sid == _NS - 1)
            def _():
                pltpu.sync_copy(zcol.at[pl.ds(0, _RL + 16)],
                                deg_sh.at[pl.ds(15 * _RF, _RL + 16)])

        plsc.subcore_barrier()

        _QC = _CPT // 4
        for q in range(4):
            pltpu.sync_copy(src_hbm.at[pl.ds(crow + q * _QC, _QC), :], srcl)
            pltpu.sync_copy(dst_hbm.at[pl.ds(crow + q * _QC, _QC), :], dstl)
            for b in range(_LA):
                pltpu.async_copy(tab_sh.at[srcl.at[b]], ring[b], gsems[b])

            def step(i, carry):
                for b in range(_NB):
                    cc = i * _NB + b
                    pltpu.make_async_copy(tab_sh.at[srcl.at[0]], ring[b],
                                          gsems[b]).wait()
                    pltpu.async_copy(ring[b], agg_sh.at[dstl.at[cc]],
                                     ssems[b], add=True)
                    if with_deg:
                        pltpu.async_copy(onesv, deg_sh.at[dstl.at[cc]],
                                         dsem, add=True)

                        @pl.when(cc >= _NB)
                        def _():
                            pltpu.make_async_copy(
                                onesv, deg_sh.at[dstl.at[0]], dsem).wait()
                    nc = cc + _LA
                    nb = (b + _LA) % _NB

                    @pl.when(cc >= _NB - _LA)
                    def _():
                        pltpu.make_async_copy(ring[nb],
                                              agg_sh.at[dstl.at[0]],
                                              ssems[nb]).wait()

                    @pl.when(nc < _QC)
                    def _():
                        pltpu.async_copy(tab_sh.at[srcl.at[nc]], ring[nb],
                                         gsems[nb])
                return carry
            lax.fori_loop(0, _QC // _NB, step, 0)
            for k in range(_LA):
                b = (_QC - _LA + k) % _NB
                pltpu.make_async_copy(ring[b], agg_sh.at[dstl.at[0]],
                                      ssems[b]).wait()
            if with_deg:
                for _k in range(_NB):
                    pltpu.make_async_copy(onesv, deg_sh.at[dstl.at[0]],
                                          dsem).wait()

        plsc.subcore_barrier()

        @pl.when(sid < _NS - 1)
        def _():
            pltpu.sync_copy(agg_sh.at[pl.ds(base, _RF)],
                            out_hbm.at[cid, pl.ds(base, _RF), :])

        @pl.when(sid == _NS - 1)
        def _():
            pltpu.sync_copy(agg_sh.at[pl.ds(15 * _RF, _RL)],
                            out_hbm.at[cid, pl.ds(15 * _RF, _RL), :])

        if with_deg:
            @pl.when(sid < _NS - 1)
            def _():
                pltpu.sync_copy(deg_sh.at[pl.ds(base, _RF)],
                                deg_hbm.at[cid, pl.ds(base, _RF)])

            @pl.when(sid == _NS - 1)
            def _():
                pltpu.sync_copy(deg_sh.at[pl.ds(15 * _RF, _RL)],
                                deg_hbm.at[cid, pl.ds(15 * _RF, _RL)])

    return pl.kernel(body, out_type=out_type, mesh=mesh,
                     scratch_types=scratch,
                     compiler_params=pltpu.CompilerParams(
                         use_tc_tiling_on_sc=False))


_SC_AGG_DEG = _make_sc_agg(True)
_SC_AGG = _make_sc_agg(False)

_BN = 1000


def _make_dense(relu: bool, split_out: bool):
    def body(p0, p1, dt, x0r, x1r, wlr, wrr, br, *outs):
        deg = 0.5 * jnp.sum(dt[...], axis=1, keepdims=True)
        inv = 1.0 / jnp.maximum(deg, 1.0)
        agg = jnp.concatenate([p0[0, :, :], p1[0, :, :]], axis=1) * inv
        xr = jnp.concatenate([x0r[...], x1r[...]], axis=1)
        h = (jnp.dot(agg, wlr[...], preferred_element_type=jnp.float32)
             + jnp.dot(xr, wrr[...], preferred_element_type=jnp.float32)
             + br[...])
        if relu:
            h = jnp.maximum(h, 0.0)
        h = h + xr
        if split_out:
            outs[0][...] = h[:, :_H]
            outs[1][...] = h[:, _H:]
        else:
            outs[0][...] = h

    if split_out:
        out_shape = [jax.ShapeDtypeStruct((_N, _H), jnp.float32)] * 2
        out_specs = [pl.BlockSpec((_BN, _H), lambda i: (i, 0))] * 2
    else:
        out_shape = jax.ShapeDtypeStruct((_N, _D), jnp.float32)
        out_specs = pl.BlockSpec((_BN, _D), lambda i: (i, 0))
    return pl.pallas_call(
        body,
        grid=(_N // _BN,),
        in_specs=[
            pl.BlockSpec((1, _BN, _H), lambda i: (0, i, 0)),
            pl.BlockSpec((1, _BN, _H), lambda i: (1, i, 0)),
            pl.BlockSpec((_BN, _NC), lambda i: (i, 0)),
            pl.BlockSpec((_BN, _H), lambda i: (i, 0)),
            pl.BlockSpec((_BN, _H), lambda i: (i, 0)),
            pl.BlockSpec((_D, _D), lambda i: (0, 0)),
            pl.BlockSpec((_D, _D), lambda i: (0, 0)),
            pl.BlockSpec((1, _D), lambda i: (0, 0)),
        ],
        out_specs=out_specs,
        out_shape=out_shape,
    )


_DENSE_RELU = _make_dense(True, True)
_DENSE_LAST = _make_dense(False, False)


def kernel(x, edge_index, W_l0, W_r0, b0, W_l1, W_r1, b1, W_l2, W_r2, b2):
    npad = _EP - _E
    src2d = jnp.concatenate(
        [edge_index[0], jnp.zeros((npad,), jnp.int32)]).reshape(_CH, _KC)
    dst2d = jnp.concatenate(
        [edge_index[1], jnp.full((npad,), _N, jnp.int32)]).reshape(_CH, _KC)
    x0 = x[:, :_H]
    x1 = x[:, _H:]
    agg, degp = _SC_AGG_DEG(x0, x1, src2d, dst2d)
    degp_t = degp.T
    h0, h1 = _DENSE_RELU(agg, agg, degp_t, x0, x1, W_l0, W_r0,
                         b0.reshape(1, _D))
    agg, = _SC_AGG(h0, h1, src2d, dst2d)
    h0, h1 = _DENSE_RELU(agg, agg, degp_t, h0, h1, W_l1, W_r1,
                         b1.reshape(1, _D))
    agg, = _SC_AGG(h0, h1, src2d, dst2d)
    h = _DENSE_LAST(agg, agg, degp_t, h0, h1, W_l2, W_r2,
                    b2.reshape(1, _D))
    return h
